# baseline (speedup 1.0000x reference)
import jax
import jax.numpy as jnp
import numpy as np
from jax import lax
from jax.experimental import pallas as pl
from jax.experimental.pallas import tpu as pltpu

_B, _K, _H, _W = 64, 17, 128, 128
_V = _H * _W
_G = 8
_R = _B * _K

_TINY = np.float32(np.finfo(np.float32).tiny)
_EPS = np.float32(1e-8)
_KS0 = np.int32(0)
_KS1 = np.int32(42)
_KS2 = np.int32(0 ^ 42 ^ 0x1BD11BDA)
_ROT_A = (13, 15, 26, 6)
_ROT_B = (17, 29, 16, 24)


def _rotl(x, d):
    return lax.shift_left(x, np.int32(d)) | lax.shift_right_logical(
        x, np.int32(32 - d)
    )


def _four_rounds(x0, x1, rots):
    for r in rots:
        x0 = x0 + x1
        x1 = _rotl(x1, r)
        x1 = x0 ^ x1
    return x0, x1


def _threefry_bits(p):
    x0 = jnp.zeros_like(p) + _KS0
    x1 = p + _KS1
    x0, x1 = _four_rounds(x0, x1, _ROT_A)
    x0 = x0 + _KS1
    x1 = x1 + _KS2 + np.int32(1)
    x0, x1 = _four_rounds(x0, x1, _ROT_B)
    x0 = x0 + _KS2
    x1 = x1 + _KS0 + np.int32(2)
    x0, x1 = _four_rounds(x0, x1, _ROT_A)
    x0 = x0 + _KS0
    x1 = x1 + _KS1 + np.int32(3)
    x0, x1 = _four_rounds(x0, x1, _ROT_B)
    x0 = x0 + _KS1
    x1 = x1 + _KS2 + np.int32(4)
    x0, x1 = _four_rounds(x0, x1, _ROT_A)
    x0 = x0 + _KS2
    x1 = x1 + _KS0 + np.int32(5)
    return x0 ^ x1


def _sample_body(hm_ref, idx_ref, logp_ref):
    b = pl.program_id(0)
    k = pl.program_id(1)
    l = hm_ref[0, 0]

    m = jnp.max(l)
    lse = jnp.log(jnp.sum(jnp.exp(l - m)))

    row = b * np.int32(_K) + k
    vi = (
        lax.broadcasted_iota(jnp.int32, (_H, _W), 0) * np.int32(_W)
        + lax.broadcasted_iota(jnp.int32, (_H, _W), 1)
    )
    lane = lax.broadcasted_iota(jnp.int32, (1, 1, 1, _G), 3)
    idx_out = jnp.zeros((1, 1, 1, _G), jnp.int32)
    logp_out = jnp.zeros((1, 1, 1, _G), jnp.float32)
    for s in range(_G):
        base = (np.int32(s * _R) + row) * np.int32(_V)
        bits = _threefry_bits(base + vi)
        fbits = lax.shift_right_logical(bits, np.int32(9)) | np.int32(0x3F800000)
        f = lax.bitcast_convert_type(fbits, jnp.float32) - np.float32(1.0)
        u = jnp.maximum(_TINY, f + _TINY)
        z = -jnp.log(-jnp.log(u)) + l
        zm = jnp.max(z)
        win = jnp.min(jnp.where(z == zm, vi, np.int32(_V)))
        lwin = jnp.sum(jnp.where(vi == win, l, np.float32(0.0)))
        idx_out = jnp.where(lane == s, win, idx_out)
        logp_out = jnp.where(lane == s, (lwin - m) - lse, logp_out)
    idx_ref[...] = idx_out
    logp_ref[...] = logp_out


def _loss_body(idx_ref, logp_ref, out_ref):
    idx = idx_ref[...]
    logp = logp_ref[...]
    x = (idx % np.int32(_W)).astype(jnp.float32)
    y = (idx // np.int32(_W)).astype(jnp.float32)
    cx = np.float32((_W - 1) / 2.0)
    cy = np.float32((_H - 1) / 2.0)
    d = jnp.sqrt((x - cx) * (x - cx) + (y - cy) * (y - cy))
    rewards = -(jnp.sum(d, axis=0) / np.float32(_K)) / np.float32(max(_H, _W))
    rmean = jnp.mean(rewards, axis=-1, keepdims=True)
    dev = rewards - rmean
    std = jnp.sqrt(jnp.sum(dev * dev, axis=-1, keepdims=True) / np.float32(_G - 1))
    adv = dev / jnp.maximum(std, _EPS)
    adv = jnp.clip(adv, -5.0, 5.0)
    log_pi = jnp.sum(logp, axis=0)
    loss = -jnp.mean(adv * log_pi)
    reward_mean = jnp.mean(rewards)
    rdev = rewards - reward_mean
    reward_std = jnp.sqrt(jnp.sum(rdev * rdev) / np.float32(_B * _G - 1))
    adv_abs_mean = jnp.mean(jnp.abs(adv))
    lanes = lax.broadcasted_iota(jnp.int32, (1, 128), 1)
    vec = jnp.where(lanes == 0, loss, np.float32(0.0))
    vec = jnp.where(lanes == 1, reward_mean, vec)
    vec = jnp.where(lanes == 2, reward_std, vec)
    vec = jnp.where(lanes == 3, adv_abs_mean, vec)
    out_ref[...] = vec


def _run(heatmaps, interpret=False):
    idx, logp = pl.pallas_call(
        _sample_body,
        grid=(_B, _K),
        in_specs=[
            pl.BlockSpec((1, 1, _H, _W), lambda b, k: (b, k, 0, 0)),
        ],
        out_specs=[
            pl.BlockSpec((1, 1, 1, _G), lambda b, k: (k, b, 0, 0)),
            pl.BlockSpec((1, 1, 1, _G), lambda b, k: (k, b, 0, 0)),
        ],
        out_shape=[
            jax.ShapeDtypeStruct((_K, _B, 1, _G), jnp.int32),
            jax.ShapeDtypeStruct((_K, _B, 1, _G), jnp.float32),
        ],
        compiler_params=pltpu.CompilerParams(
            dimension_semantics=("parallel", "parallel")
        ),
        interpret=interpret,
    )(heatmaps)

    out = pl.pallas_call(
        _loss_body,
        in_specs=[
            pl.BlockSpec(idx.shape, lambda: (0, 0, 0, 0)),
            pl.BlockSpec(logp.shape, lambda: (0, 0, 0, 0)),
        ],
        out_specs=pl.BlockSpec((1, 128), lambda: (0, 0)),
        out_shape=jax.ShapeDtypeStruct((1, 128), jnp.float32),
        interpret=interpret,
    )(idx, logp)
    return (out[0, 0], out[0, 1], out[0, 2], out[0, 3])


def kernel(heatmaps):
    return _run(heatmaps)
